# Initial kernel scaffold; baseline (speedup 1.0000x reference)
import jax
import jax.numpy as jnp
from jax.experimental import pallas as pl


def kernel(boxes, scores):
    raise NotImplementedError("write your pallas kernel here")


# speedup vs baseline: 27.1915x; 27.1915x over previous
import functools

import jax
import jax.numpy as jnp
from jax import lax
from jax.experimental import pallas as pl
from jax.experimental.pallas import tpu as pltpu
from jax.experimental.pallas import tpu_sc as plsc

N = 20000
NPAD = 20096
NUM_CLASSES = 20
MAX_PER_CLASS = 100
MAX_PER_IMAGE = 300
IOU_THRESH = 0.7
NEG = -1e9
PAD = -2e9
GONE = -3e9
BIGI = jnp.int32(1 << 30)

NCHUNK = NPAD // 16
NL1 = 1280
NL2 = 80
SELPAD = 112
WALK_BUDGET = 160

NB = NUM_CLASSES * 128
NBCHUNK = NB // 16
NBL1 = 160
BB_OUT = 1280
CL_OUT = 384

_MESH = plsc.VectorSubcoreMesh(
    core_axis_name="c", subcore_axis_name="s", num_cores=2, num_subcores=16
)


def _lane():
  return lax.iota(jnp.int32, 16)


def _bfly_max(v):
  lane = _lane()
  for sh in (8, 4, 2, 1):
    perm = lax.bitwise_xor(lane, jnp.int32(sh))
    v = jnp.maximum(v, v.at[perm].get(mode="promise_in_bounds"))
  return v


def _bfly_min(v):
  lane = _lane()
  for sh in (8, 4, 2, 1):
    perm = lax.bitwise_xor(lane, jnp.int32(sh))
    v = jnp.minimum(v, v.at[perm].get(mode="promise_in_bounds"))
  return v


def _first_eq(vec, m, base_idx):
  cand = jnp.where(vec == m, base_idx + _lane(), BIGI)
  return _bfly_min(cand)[0]


def _eread(ref, base, off):
  ch = ref[pl.ds(base, 16)]
  return ch.at[jnp.broadcast_to(off, (16,))].get(mode="promise_in_bounds")


def _ewrite(ref, base, tgt_lane, val):
  ch = ref[pl.ds(base, 16)]
  ref[pl.ds(base, 16)] = jnp.where(_lane() == tgt_lane, val, ch)


def _chunk_max_splat(ref, c):
  return _bfly_max(ref[pl.ds(c * 16, 16)])


@functools.partial(
    pl.kernel,
    out_type=[
        jax.ShapeDtypeStruct((NUM_CLASSES, 128), jnp.float32),
        jax.ShapeDtypeStruct((NUM_CLASSES, 128), jnp.int32),
    ],
    mesh=_MESH,
    scratch_types=[
        pltpu.VMEM((NPAD,), jnp.float32),
        pltpu.VMEM((NPAD,), jnp.float32),
        pltpu.VMEM((NPAD,), jnp.float32),
        pltpu.VMEM((NPAD,), jnp.float32),
        pltpu.VMEM((NPAD,), jnp.float32),
        pltpu.VMEM((NL1,), jnp.float32),
        pltpu.VMEM((NL2,), jnp.float32),
        pltpu.VMEM((SELPAD,), jnp.float32),
        pltpu.VMEM((SELPAD,), jnp.float32),
        pltpu.VMEM((SELPAD,), jnp.float32),
        pltpu.VMEM((SELPAD,), jnp.float32),
        pltpu.VMEM((SELPAD,), jnp.float32),
        pltpu.VMEM((128,), jnp.float32),
        pltpu.VMEM((128,), jnp.int32),
    ],
)
def _nms_phase(boxes_t, scores_t, out_sc, out_ix,
               y1v, x1v, y2v, x2v, scv, l1v, l2v,
               sy1, sx1, sy2, sx2, sar, osc, oix):
  wid = lax.axis_index("s") * 2 + lax.axis_index("c")
  lane = _lane()

  def reset_selected():
    zerov = jnp.zeros((16,), jnp.float32)
    for k in range(SELPAD // 16):
      sy1[pl.ds(k * 16, 16)] = zerov
      sx1[pl.ds(k * 16, 16)] = zerov
      sy2[pl.ds(k * 16, 16)] = zerov
      sx2[pl.ds(k * 16, 16)] = zerov
      sar[pl.ds(k * 16, 16)] = zerov

  def build_hierarchy():
    negv = jnp.full((16,), NEG, jnp.float32)
    for k in range(NL1 // 16):
      l1v[pl.ds(k * 16, 16)] = negv

    def build1(i, _):
      _ewrite(l1v, lax.shift_left(lax.shift_right_logical(i, 4), 4),
              i & 15, _chunk_max_splat(scv, i))
      return 0
    lax.fori_loop(0, NCHUNK, build1, 0)

    def build2(j, _):
      _ewrite(l2v, lax.shift_left(lax.shift_right_logical(j, 4), 4),
              j & 15, _chunk_max_splat(l1v, j))
      return 0
    lax.fori_loop(0, NL1 // 16, build2, 0)

  def pop_top():
    t = l2v[pl.ds(0, 16)]
    for k in range(1, NL2 // 16):
      t = jnp.maximum(t, l2v[pl.ds(k * 16, 16)])
    m = _bfly_max(t)
    ms = m[0]
    j2acc = jnp.full((16,), BIGI, jnp.int32)
    for k in range(NL2 // 16):
      j2acc = jnp.minimum(
          j2acc, jnp.where(l2v[pl.ds(k * 16, 16)] == m, k * 16 + lane, BIGI))
    j2 = _bfly_min(j2acc)[0]
    c = _first_eq(l1v[pl.ds(j2 * 16, 16)], m, j2 * 16)
    idx = _first_eq(scv[pl.ds(c * 16, 16)], m, c * 16)
    return m, ms, j2, c, idx

  def load_box(idx):
    base = lax.shift_left(lax.shift_right_logical(idx, 4), 4)
    off = idx & 15
    by1 = _eread(y1v, base, off)
    bx1 = _eread(x1v, base, off)
    by2 = _eread(y2v, base, off)
    bx2 = _eread(x2v, base, off)
    return by1, bx1, by2, bx2, (by2 - by1) * (bx2 - bx1)

  def max_iou_vs_selected(by1, bx1, by2, bx2, barea):
    def iou_body(j, accf):
      ty1 = jnp.maximum(by1, sy1[pl.ds(j * 16, 16)])
      tx1 = jnp.maximum(bx1, sx1[pl.ds(j * 16, 16)])
      ty2 = jnp.minimum(by2, sy2[pl.ds(j * 16, 16)])
      tx2 = jnp.minimum(bx2, sx2[pl.ds(j * 16, 16)])
      inter = jnp.maximum(ty2 - ty1, 0.0) * jnp.maximum(tx2 - tx1, 0.0)
      iou = inter / (barea + sar[pl.ds(j * 16, 16)] - inter + 1e-8)
      return jnp.maximum(accf, iou)
    accf = lax.fori_loop(0, SELPAD // 16, iou_body,
                         jnp.zeros((16,), jnp.float32))
    return _bfly_max(accf)[0]

  def mark_and_fix(c, j2, idx, active):
    tgt = jnp.where(active, idx & 15, jnp.int32(16))
    _ewrite(scv, c * 16, tgt, jnp.float32(NEG))
    tgt1 = jnp.where(active, c & 15, jnp.int32(16))
    _ewrite(l1v, lax.shift_left(lax.shift_right_logical(c, 4), 4), tgt1,
            _chunk_max_splat(scv, c))
    tgt2 = jnp.where(active, j2 & 15, jnp.int32(16))
    _ewrite(l2v, lax.shift_left(lax.shift_right_logical(j2, 4), 4), tgt2,
            _chunk_max_splat(l1v, j2))

  def append(ns, rec, vsc, vix, bxs, add_sel):
    base = lax.shift_left(lax.shift_right_logical(ns, 4), 4)
    tgt = jnp.where(rec, ns & 15, jnp.int32(16))
    _ewrite(osc, base, tgt, vsc)
    _ewrite(oix, base, tgt, vix)
    tgts = jnp.where(add_sel, ns & 15, jnp.int32(16))
    by1, bx1, by2, bx2, barea = bxs
    _ewrite(sy1, base, tgts, by1)
    _ewrite(sx1, base, tgts, bx1)
    _ewrite(sy2, base, tgts, by2)
    _ewrite(sx2, base, tgts, bx2)
    _ewrite(sar, base, tgts, barea)

  @pl.when(wid < NUM_CLASSES)
  def _():
    cls = wid
    pltpu.sync_copy(boxes_t.at[0], y1v)
    pltpu.sync_copy(boxes_t.at[1], x1v)
    pltpu.sync_copy(boxes_t.at[2], y2v)
    pltpu.sync_copy(boxes_t.at[3], x2v)
    pltpu.sync_copy(scores_t.at[cls], scv)

    padv = jnp.full((16,), PAD, jnp.float32)
    zeroiv = jnp.zeros((16,), jnp.int32)
    for k in range(8):
      osc[pl.ds(k * 16, 16)] = padv
      oix[pl.ds(k * 16, 16)] = zeroiv
    reset_selected()
    build_hierarchy()

    def walk_body(_, ns):
      m, ms, j2, c, idx = pop_top()
      valid = ms > jnp.float32(-0.5)
      done = ns < MAX_PER_CLASS
      active = jnp.logical_and(valid, done)
      bxs = load_box(idx)
      miou = max_iou_vs_selected(*bxs)
      accept = jnp.logical_and(active, miou <= IOU_THRESH)
      mark_and_fix(c, j2, idx, active)
      rec = jnp.logical_and(done, jnp.logical_or(accept,
                                                 jnp.logical_not(valid)))
      append(ns, rec, m, idx, bxs, accept)
      return ns + jnp.where(rec, 1, 0)

    ns = lax.fori_loop(0, WALK_BUDGET, walk_body, jnp.int32(0))

    @pl.when(ns < MAX_PER_CLASS)
    def _():
      pltpu.sync_copy(scores_t.at[cls], scv)
      reset_selected()
      build_hierarchy()

      def eager_body(step, _):
        m, ms, j2, c, idx = pop_top()
        valid = ms > jnp.float32(-0.5)
        bxs = load_box(idx)
        by1, bx1, by2, bx2, barea = bxs
        append(step, True, m, idx, bxs, valid)

        def sweep(i, _):
          v = scv[pl.ds(i * 16, 16)]
          ty1 = jnp.maximum(by1, y1v[pl.ds(i * 16, 16)])
          tx1 = jnp.maximum(bx1, x1v[pl.ds(i * 16, 16)])
          ty2 = jnp.minimum(by2, y2v[pl.ds(i * 16, 16)])
          tx2 = jnp.minimum(bx2, x2v[pl.ds(i * 16, 16)])
          oy1 = y1v[pl.ds(i * 16, 16)]
          oarea = ((y2v[pl.ds(i * 16, 16)] - oy1) *
                   (x2v[pl.ds(i * 16, 16)] - x1v[pl.ds(i * 16, 16)]))
          inter = jnp.maximum(ty2 - ty1, 0.0) * jnp.maximum(tx2 - tx1, 0.0)
          iou = inter / (barea + oarea - inter + 1e-8)
          v = jnp.where(iou > IOU_THRESH, jnp.float32(NEG), v)
          v = jnp.where(i * 16 + _lane() == idx, jnp.float32(NEG), v)
          scv[pl.ds(i * 16, 16)] = v
          _ewrite(l1v, lax.shift_left(lax.shift_right_logical(i, 4), 4),
                  i & 15, _bfly_max(v))
          return 0
        lax.fori_loop(0, NCHUNK, sweep, 0)

        def rebuild2(j, _):
          _ewrite(l2v, lax.shift_left(lax.shift_right_logical(j, 4), 4),
                  j & 15, _chunk_max_splat(l1v, j))
          return 0
        lax.fori_loop(0, NL1 // 16, rebuild2, 0)
        return 0

      lax.fori_loop(0, MAX_PER_CLASS, eager_body, 0)

    pltpu.sync_copy(osc, out_sc.at[cls])
    pltpu.sync_copy(oix, out_ix.at[cls])


@functools.partial(
    pl.kernel,
    out_type=[
        jax.ShapeDtypeStruct((BB_OUT,), jnp.float32),
        jax.ShapeDtypeStruct((CL_OUT,), jnp.int32),
    ],
    mesh=_MESH,
    scratch_types=[
        pltpu.VMEM((NB,), jnp.float32),
        pltpu.VMEM((NB,), jnp.int32),
        pltpu.VMEM((NPAD,), jnp.float32),
        pltpu.VMEM((NPAD,), jnp.float32),
        pltpu.VMEM((NPAD,), jnp.float32),
        pltpu.VMEM((NPAD,), jnp.float32),
        pltpu.VMEM((NBL1,), jnp.float32),
        pltpu.VMEM((BB_OUT,), jnp.float32),
        pltpu.VMEM((CL_OUT,), jnp.int32),
    ],
)
def _topk_phase(flat_sc_h, flat_ix_h, boxes_t, out_bb, out_cl,
                fsc, fix, y1v, x1v, y2v, x2v, l1v, bbs, cls_s):
  wid = lax.axis_index("s") * 2 + lax.axis_index("c")
  lane = _lane()

  @pl.when(wid == 0)
  def _():
    pltpu.sync_copy(flat_sc_h, fsc)
    pltpu.sync_copy(flat_ix_h, fix)
    pltpu.sync_copy(boxes_t.at[0], y1v)
    pltpu.sync_copy(boxes_t.at[1], x1v)
    pltpu.sync_copy(boxes_t.at[2], y2v)
    pltpu.sync_copy(boxes_t.at[3], x2v)

    zf = jnp.zeros((16,), jnp.float32)
    zi = jnp.zeros((16,), jnp.int32)
    for k in range(BB_OUT // 16):
      bbs[pl.ds(k * 16, 16)] = zf
    for k in range(CL_OUT // 16):
      cls_s[pl.ds(k * 16, 16)] = zi

    def build1(i, _):
      _ewrite(l1v, lax.shift_left(lax.shift_right_logical(i, 4), 4),
              i & 15, _chunk_max_splat(fsc, i))
      return 0
    lax.fori_loop(0, NBCHUNK, build1, 0)

    def body(r, _):
      t = l1v[pl.ds(0, 16)]
      for k in range(1, NBL1 // 16):
        t = jnp.maximum(t, l1v[pl.ds(k * 16, 16)])
      m = _bfly_max(t)
      cacc = jnp.full((16,), BIGI, jnp.int32)
      for k in range(NBL1 // 16):
        cacc = jnp.minimum(
            cacc, jnp.where(l1v[pl.ds(k * 16, 16)] == m, k * 16 + lane, BIGI))
      c = _bfly_min(cacc)[0]
      fidx = _first_eq(fsc[pl.ds(c * 16, 16)], m, c * 16)

      fch = fix[pl.ds(c * 16, 16)]
      bsel = jnp.where(lane == (fidx & 15), fch, jnp.int32(-1))
      bi = _bfly_max(bsel)[0]
      klass = lax.shift_right_logical(fidx, 7)

      bbase = lax.shift_left(lax.shift_right_logical(bi, 4), 4)
      boff = bi & 15
      by1 = _eread(y1v, bbase, boff)
      bx1 = _eread(x1v, bbase, boff)
      by2 = _eread(y2v, bbase, boff)
      bx2 = _eread(x2v, bbase, boff)

      obase = lax.shift_left(lax.shift_right_logical(r * 4, 4), 4)
      o = (r * 4) & 15
      och = bbs[pl.ds(obase, 16)]
      och = jnp.where(lane == o, by1, och)
      och = jnp.where(lane == o + 1, bx1, och)
      och = jnp.where(lane == o + 2, by2, och)
      och = jnp.where(lane == o + 3, bx2, och)
      bbs[pl.ds(obase, 16)] = och
      _ewrite(cls_s, lax.shift_left(lax.shift_right_logical(r, 4), 4),
              r & 15, klass)

      _ewrite(fsc, c * 16, fidx & 15, jnp.float32(GONE))
      _ewrite(l1v, lax.shift_left(lax.shift_right_logical(c, 4), 4),
              c & 15, _chunk_max_splat(fsc, c))
      return 0

    lax.fori_loop(0, MAX_PER_IMAGE, body, 0)

    pltpu.sync_copy(bbs, out_bb)
    pltpu.sync_copy(cls_s, out_cl)


def kernel(boxes, scores):
  boxes_p = jnp.pad(boxes, ((0, NPAD - N), (0, 0)))
  scores_p = jnp.pad(scores, ((0, NPAD - N), (0, 0)), constant_values=NEG)
  boxes_t = boxes_p.T
  scores_t = scores_p.T
  sc_a, ix_a = _nms_phase(boxes_t, scores_t)
  bb_flat, cl = _topk_phase(sc_a.reshape(-1), ix_a.reshape(-1), boxes_t)
  return (bb_flat[:MAX_PER_IMAGE * 4].reshape(MAX_PER_IMAGE, 4),
          cl[:MAX_PER_IMAGE])


# speedup vs baseline: 39.2031x; 1.4417x over previous
import functools

import jax
import jax.numpy as jnp
from jax import lax
from jax.experimental import pallas as pl
from jax.experimental.pallas import tpu as pltpu
from jax.experimental.pallas import tpu_sc as plsc

N = 20000
NPAD = 20096
NSCV = 20480
NUM_CLASSES = 20
MAX_PER_CLASS = 100
MAX_PER_IMAGE = 300
IOU_THRESH = 0.7
NEG = -1e9
PAD = -2e9
LOW = -3e9
BIGI = jnp.int32(1 << 30)

NCHUNK = NPAD // 16
NL1 = 1280
NL2 = 80
SELPAD = 112
WALK_BLOCKS = 14

NB = NUM_CLASSES * 128
BB_OUT = 1280
CL_OUT = 384

_MESH = plsc.VectorSubcoreMesh(
    core_axis_name="c", subcore_axis_name="s", num_cores=2, num_subcores=16
)


def _lane():
  return lax.iota(jnp.int32, 16)


def _bfly_max(v):
  lane = _lane()
  for sh in (8, 4, 2, 1):
    perm = lax.bitwise_xor(lane, jnp.int32(sh))
    v = jnp.maximum(v, v.at[perm].get(mode="promise_in_bounds"))
  return v


def _bfly_min(v):
  lane = _lane()
  for sh in (8, 4, 2, 1):
    perm = lax.bitwise_xor(lane, jnp.int32(sh))
    v = jnp.minimum(v, v.at[perm].get(mode="promise_in_bounds"))
  return v


def _first_eq_vec(vec, m, base_idx):
  cand = jnp.where(vec == m, base_idx + _lane(), BIGI)
  return _bfly_min(cand)[0]


def _eread(ref, base, off):
  ch = ref[pl.ds(base, 16)]
  return ch.at[jnp.broadcast_to(off, (16,))].get(mode="promise_in_bounds")


def _ewrite(ref, base, tgt_lane, val):
  ch = ref[pl.ds(base, 16)]
  ref[pl.ds(base, 16)] = jnp.where(_lane() == tgt_lane, val, ch)


def _align16(i):
  return lax.shift_left(lax.shift_right_logical(i, 4), 4)


@functools.partial(
    pl.kernel,
    out_type=[
        jax.ShapeDtypeStruct((NUM_CLASSES, 128), jnp.float32),
        jax.ShapeDtypeStruct((NUM_CLASSES, 128), jnp.int32),
    ],
    mesh=_MESH,
    scratch_types=[
        pltpu.VMEM((NPAD,), jnp.float32),
        pltpu.VMEM((NPAD,), jnp.float32),
        pltpu.VMEM((NPAD,), jnp.float32),
        pltpu.VMEM((NPAD,), jnp.float32),
        pltpu.VMEM((NSCV,), jnp.float32),
        pltpu.VMEM((NL1,), jnp.float32),
        pltpu.VMEM((NL2,), jnp.float32),
        pltpu.VMEM((SELPAD,), jnp.float32),
        pltpu.VMEM((SELPAD,), jnp.float32),
        pltpu.VMEM((SELPAD,), jnp.float32),
        pltpu.VMEM((SELPAD,), jnp.float32),
        pltpu.VMEM((SELPAD,), jnp.float32),
        pltpu.VMEM((128,), jnp.float32),
        pltpu.VMEM((128,), jnp.int32),
        pltpu.VMEM((16,), jnp.int32),
    ],
)
def _nms_phase(boxes_t, scores_t, out_sc, out_ix,
               y1v, x1v, y2v, x2v, scv, l1v, l2v,
               sy1, sx1, sy2, sx2, sar, osc, oix, ncell):
  wid = lax.axis_index("s") * 2 + lax.axis_index("c")
  lane = _lane()

  def reset_selected():
    zerov = jnp.zeros((16,), jnp.float32)
    for k in range(SELPAD // 16):
      sy1[pl.ds(k * 16, 16)] = zerov
      sx1[pl.ds(k * 16, 16)] = zerov
      sy2[pl.ds(k * 16, 16)] = zerov
      sx2[pl.ds(k * 16, 16)] = zerov
      sar[pl.ds(k * 16, 16)] = zerov

  def build_hierarchy():
    def build1g(g, _):
      acc = jnp.full((16,), NEG, jnp.float32)
      base = lax.shift_left(g, 8)
      for kk in range(16):
        bm = _bfly_max(scv[pl.ds(base + kk * 16, 16)])
        acc = jnp.where(lane == kk, bm, acc)
      l1v[pl.ds(g * 16, 16)] = acc
      return 0
    lax.fori_loop(0, NL1 // 16, build1g, 0)

    def build2g(g, _):
      acc = jnp.full((16,), NEG, jnp.float32)
      base = lax.shift_left(g, 8)
      for kk in range(16):
        bm = _bfly_max(l1v[pl.ds(base + kk * 16, 16)])
        acc = jnp.where(lane == kk, bm, acc)
      l2v[pl.ds(g * 16, 16)] = acc
      return 0
    lax.fori_loop(0, NL2 // 16, build2g, 0)

  def pop_top():
    l2regs = [l2v[pl.ds(k * 16, 16)] for k in range(NL2 // 16)]
    t = l2regs[0]
    for k in range(1, NL2 // 16):
      t = jnp.maximum(t, l2regs[k])
    m = _bfly_max(t)
    ms = m[0]
    j2acc = jnp.full((16,), BIGI, jnp.int32)
    for k in range(NL2 // 16):
      j2acc = jnp.minimum(
          j2acc, jnp.where(l2regs[k] == m, k * 16 + lane, BIGI))
    j2 = _bfly_min(j2acc)[0]
    l1ch = l1v[pl.ds(j2 * 16, 16)]
    c = _first_eq_vec(l1ch, m, j2 * 16)
    leafch = scv[pl.ds(c * 16, 16)]
    idx = _first_eq_vec(leafch, m, c * 16)
    return m, ms, j2, c, idx, l1ch, leafch

  def load_box(idx):
    base = _align16(idx)
    off = idx & 15
    by1 = _eread(y1v, base, off)
    bx1 = _eread(x1v, base, off)
    by2 = _eread(y2v, base, off)
    bx2 = _eread(x2v, base, off)
    return by1, bx1, by2, bx2, (by2 - by1) * (bx2 - bx1)

  def max_iou_vs_selected(by1, bx1, by2, bx2, barea):
    def iou_body(j, accf):
      ty1 = jnp.maximum(by1, sy1[pl.ds(j * 16, 16)])
      tx1 = jnp.maximum(bx1, sx1[pl.ds(j * 16, 16)])
      ty2 = jnp.minimum(by2, sy2[pl.ds(j * 16, 16)])
      tx2 = jnp.minimum(bx2, sx2[pl.ds(j * 16, 16)])
      inter = jnp.maximum(ty2 - ty1, 0.0) * jnp.maximum(tx2 - tx1, 0.0)
      iou = inter / (barea + sar[pl.ds(j * 16, 16)] - inter + 1e-8)
      return jnp.maximum(accf, iou)
    accf = lax.fori_loop(0, SELPAD // 16, iou_body,
                         jnp.zeros((16,), jnp.float32))
    return _bfly_max(accf)[0]

  def mark_and_fix(c, j2, idx, active, l1ch, leafch):
    tgt = jnp.where(active, idx & 15, jnp.int32(16))
    newleaf = jnp.where(lane == tgt, jnp.float32(NEG), leafch)
    scv[pl.ds(c * 16, 16)] = newleaf
    tgt1 = jnp.where(active, c & 15, jnp.int32(16))
    newl1 = jnp.where(lane == tgt1, _bfly_max(newleaf), l1ch)
    l1v[pl.ds(j2 * 16, 16)] = newl1
    tgt2 = jnp.where(active, j2 & 15, jnp.int32(16))
    _ewrite(l2v, _align16(j2), tgt2, _bfly_max(newl1))

  def append(ns, rec, vsc, vix, bxs, add_sel):
    base = _align16(ns)
    tgt = jnp.where(rec, ns & 15, jnp.int32(16))
    _ewrite(osc, base, tgt, vsc)
    _ewrite(oix, base, tgt, vix)
    tgts = jnp.where(add_sel, ns & 15, jnp.int32(16))
    by1, bx1, by2, bx2, barea = bxs
    _ewrite(sy1, base, tgts, by1)
    _ewrite(sx1, base, tgts, bx1)
    _ewrite(sy2, base, tgts, by2)
    _ewrite(sx2, base, tgts, bx2)
    _ewrite(sar, base, tgts, barea)

  @pl.when(wid < NUM_CLASSES)
  def _():
    cls = wid
    pltpu.sync_copy(boxes_t.at[0], y1v)
    pltpu.sync_copy(boxes_t.at[1], x1v)
    pltpu.sync_copy(boxes_t.at[2], y2v)
    pltpu.sync_copy(boxes_t.at[3], x2v)
    pltpu.sync_copy(scores_t.at[cls], scv.at[pl.ds(0, NPAD)])

    negv = jnp.full((16,), NEG, jnp.float32)
    for k in range(NCHUNK, NSCV // 16):
      scv[pl.ds(k * 16, 16)] = negv
    padv = jnp.full((16,), PAD, jnp.float32)
    zeroiv = jnp.zeros((16,), jnp.int32)
    for k in range(8):
      osc[pl.ds(k * 16, 16)] = padv
      oix[pl.ds(k * 16, 16)] = zeroiv
    reset_selected()
    build_hierarchy()

    def walk_body(_, ns):
      m, ms, j2, c, idx, l1ch, leafch = pop_top()
      valid = ms > jnp.float32(-0.5)
      done = ns < MAX_PER_CLASS
      active = jnp.logical_and(valid, done)
      bxs = load_box(idx)
      miou = max_iou_vs_selected(*bxs)
      accept = jnp.logical_and(active, miou <= IOU_THRESH)
      mark_and_fix(c, j2, idx, active, l1ch, leafch)
      rec = jnp.logical_and(done, jnp.logical_or(accept,
                                                 jnp.logical_not(valid)))
      append(ns, rec, m, idx, bxs, accept)
      return ns + jnp.where(rec, 1, 0)

    ncell[pl.ds(0, 16)] = jnp.zeros((16,), jnp.int32)

    def walk_block(b, _):
      nsv = ncell[pl.ds(0, 16)]
      ns0 = nsv[0]

      @pl.when(ns0 < MAX_PER_CLASS)
      def _():
        ns = lax.fori_loop(0, 16, walk_body, ns0)
        ncell[pl.ds(0, 16)] = jnp.broadcast_to(ns, (16,))
      return 0

    lax.fori_loop(0, WALK_BLOCKS, walk_block, 0)
    ns_final = ncell[pl.ds(0, 16)][0]

    @pl.when(ns_final < MAX_PER_CLASS)
    def _():
      pltpu.sync_copy(scores_t.at[cls], scv.at[pl.ds(0, NPAD)])
      reset_selected()
      build_hierarchy()

      def eager_body(step, _):
        m, ms, j2, c, idx, l1ch, leafch = pop_top()
        valid = ms > jnp.float32(-0.5)
        bxs = load_box(idx)
        by1, bx1, by2, bx2, barea = bxs
        append(step, True, m, idx, bxs, valid)

        def sweep(i, _):
          v = scv[pl.ds(i * 16, 16)]
          ty1 = jnp.maximum(by1, y1v[pl.ds(i * 16, 16)])
          tx1 = jnp.maximum(bx1, x1v[pl.ds(i * 16, 16)])
          ty2 = jnp.minimum(by2, y2v[pl.ds(i * 16, 16)])
          tx2 = jnp.minimum(bx2, x2v[pl.ds(i * 16, 16)])
          oy1 = y1v[pl.ds(i * 16, 16)]
          oarea = ((y2v[pl.ds(i * 16, 16)] - oy1) *
                   (x2v[pl.ds(i * 16, 16)] - x1v[pl.ds(i * 16, 16)]))
          inter = jnp.maximum(ty2 - ty1, 0.0) * jnp.maximum(tx2 - tx1, 0.0)
          iou = inter / (barea + oarea - inter + 1e-8)
          v = jnp.where(iou > IOU_THRESH, jnp.float32(NEG), v)
          v = jnp.where(i * 16 + _lane() == idx, jnp.float32(NEG), v)
          scv[pl.ds(i * 16, 16)] = v
          _ewrite(l1v, _align16(i), i & 15, _bfly_max(v))
          return 0
        lax.fori_loop(0, NCHUNK, sweep, 0)

        def rebuild2(j, _):
          _ewrite(l2v, _align16(j), j & 15, _bfly_max(l1v[pl.ds(j * 16, 16)]))
          return 0
        lax.fori_loop(0, NL1 // 16, rebuild2, 0)
        return 0

      lax.fori_loop(0, MAX_PER_CLASS, eager_body, 0)

    pltpu.sync_copy(osc, out_sc.at[cls])
    pltpu.sync_copy(oix, out_ix.at[cls])


@functools.partial(
    pl.kernel,
    out_type=[
        jax.ShapeDtypeStruct((BB_OUT,), jnp.float32),
        jax.ShapeDtypeStruct((CL_OUT,), jnp.int32),
    ],
    mesh=_MESH,
    scratch_types=[
        pltpu.VMEM((NB,), jnp.float32),
        pltpu.VMEM((NB,), jnp.int32),
        pltpu.VMEM((NPAD,), jnp.float32),
        pltpu.VMEM((NPAD,), jnp.float32),
        pltpu.VMEM((NPAD,), jnp.float32),
        pltpu.VMEM((NPAD,), jnp.float32),
        pltpu.VMEM((BB_OUT,), jnp.float32),
        pltpu.VMEM((CL_OUT,), jnp.int32),
    ],
)
def _topk_phase(flat_sc_h, flat_ix_h, boxes_t, out_bb, out_cl,
                fsc, fix, y1v, x1v, y2v, x2v, bbs, cls_s):
  wid = lax.axis_index("s") * 2 + lax.axis_index("c")
  lane = _lane()

  @pl.when(wid == 0)
  def _():
    pltpu.sync_copy(flat_sc_h, fsc)
    pltpu.sync_copy(flat_ix_h, fix)
    pltpu.sync_copy(boxes_t.at[0], y1v)
    pltpu.sync_copy(boxes_t.at[1], x1v)
    pltpu.sync_copy(boxes_t.at[2], y2v)
    pltpu.sync_copy(boxes_t.at[3], x2v)

    h0 = jnp.full((16,), LOW, jnp.float32)
    h1 = jnp.full((16,), LOW, jnp.float32)
    for c in range(16):
      h0 = jnp.where(lane == c, _eread(fsc, c * 128, 0), h0)
    for c in range(16, NUM_CLASSES):
      h1 = jnp.where(lane == (c - 16), _eread(fsc, c * 128, 0), h1)
    p0 = jnp.zeros((16,), jnp.int32)
    p1 = jnp.zeros((16,), jnp.int32)

    def block(b, carry):
      h0, h1, p0, p1, clsacc = carry
      och = jnp.zeros((16,), jnp.float32)
      for s in range(4):
        r = 4 * b + s
        m = _bfly_max(jnp.maximum(h0, h1))
        c0 = jnp.where(h0 == m, lane, BIGI)
        c1 = jnp.where(h1 == m, 16 + lane, BIGI)
        cls = _bfly_min(jnp.minimum(c0, c1))[0]
        s0 = jnp.where(lane == cls, p0, BIGI)
        s1 = jnp.where(lane + 16 == cls, p1, BIGI)
        slot = _bfly_min(jnp.minimum(s0, s1))[0]
        fidx = cls * 128 + slot

        nh = _eread(fsc, _align16(fidx + 1), (fidx + 1) & 15)
        ich = fix[pl.ds(_align16(fidx), 16)]
        bsel = jnp.where(lane == (fidx & 15), ich, jnp.int32(-1))
        bi = _bfly_max(bsel)[0]

        bbase = _align16(bi)
        boff = bi & 15
        och = jnp.where(lane == 4 * s + 0, _eread(y1v, bbase, boff), och)
        och = jnp.where(lane == 4 * s + 1, _eread(x1v, bbase, boff), och)
        och = jnp.where(lane == 4 * s + 2, _eread(y2v, bbase, boff), och)
        och = jnp.where(lane == 4 * s + 3, _eread(x2v, bbase, boff), och)
        clsacc = jnp.where(lane == (r & 15), cls, clsacc)

        h0 = jnp.where(lane == cls, nh, h0)
        p0 = jnp.where(lane == cls, p0 + 1, p0)
        h1 = jnp.where(lane + 16 == cls, nh, h1)
        p1 = jnp.where(lane + 16 == cls, p1 + 1, p1)

      bbs[pl.ds(b * 16, 16)] = och
      cls_s[pl.ds(_align16(b * 4), 16)] = clsacc
      return (h0, h1, p0, p1, clsacc)

    lax.fori_loop(0, MAX_PER_IMAGE // 4, block,
                  (h0, h1, p0, p1, jnp.zeros((16,), jnp.int32)))

    pltpu.sync_copy(bbs, out_bb)
    pltpu.sync_copy(cls_s, out_cl)


def kernel(boxes, scores):
  boxes_p = jnp.pad(boxes, ((0, NPAD - N), (0, 0)))
  scores_p = jnp.pad(scores, ((0, NPAD - N), (0, 0)), constant_values=NEG)
  boxes_t = boxes_p.T
  scores_t = scores_p.T
  sc_a, ix_a = _nms_phase(boxes_t, scores_t)
  bb_flat, cl = _topk_phase(sc_a.reshape(-1), ix_a.reshape(-1), boxes_t)
  return (bb_flat[:MAX_PER_IMAGE * 4].reshape(MAX_PER_IMAGE, 4),
          cl[:MAX_PER_IMAGE])


# speedup vs baseline: 44.3182x; 1.1305x over previous
import functools

import jax
import jax.numpy as jnp
from jax import lax
from jax.experimental import pallas as pl
from jax.experimental.pallas import tpu as pltpu
from jax.experimental.pallas import tpu_sc as plsc

N = 20000
NPAD = 20096
NSCV = 20480
NUM_CLASSES = 20
MAX_PER_CLASS = 100
MAX_PER_IMAGE = 300
IOU_THRESH = 0.7
NEG = -1e9
PAD = -2e9
LOW = -3e9
BIGI = jnp.int32(1 << 30)

NCHUNK = NPAD // 16
NGRP = NSCV // 256
NA = 80
SELPAD = 112
WALK_BLOCKS = 14

NB = NUM_CLASSES * 128
BB_OUT = 1280
CL_OUT = 384

_MESH = plsc.VectorSubcoreMesh(
    core_axis_name="c", subcore_axis_name="s", num_cores=2, num_subcores=16
)


def _lane():
  return lax.iota(jnp.int32, 16)


def _bfly_max(v):
  lane = _lane()
  for sh in (8, 4, 2, 1):
    perm = lax.bitwise_xor(lane, jnp.int32(sh))
    v = jnp.maximum(v, v.at[perm].get(mode="promise_in_bounds"))
  return v


def _bfly_min(v):
  lane = _lane()
  for sh in (8, 4, 2, 1):
    perm = lax.bitwise_xor(lane, jnp.int32(sh))
    v = jnp.minimum(v, v.at[perm].get(mode="promise_in_bounds"))
  return v


def _eread(ref, base, off):
  ch = ref[pl.ds(base, 16)]
  return ch.at[jnp.broadcast_to(off, (16,))].get(mode="promise_in_bounds")


def _ewrite(ref, base, tgt_lane, val):
  ch = ref[pl.ds(base, 16)]
  ref[pl.ds(base, 16)] = jnp.where(_lane() == tgt_lane, val, ch)


def _align16(i):
  return lax.shift_left(lax.shift_right_logical(i, 4), 4)


@functools.partial(
    pl.kernel,
    out_type=[
        jax.ShapeDtypeStruct((NUM_CLASSES, 128), jnp.float32),
        jax.ShapeDtypeStruct((NUM_CLASSES, 128), jnp.int32),
    ],
    mesh=_MESH,
    scratch_types=[
        pltpu.VMEM((NPAD,), jnp.float32),
        pltpu.VMEM((NPAD,), jnp.float32),
        pltpu.VMEM((NPAD,), jnp.float32),
        pltpu.VMEM((NPAD,), jnp.float32),
        pltpu.VMEM((NSCV,), jnp.float32),
        pltpu.VMEM((NGRP * 16,), jnp.float32),
        pltpu.VMEM((NA,), jnp.float32),
        pltpu.VMEM((SELPAD,), jnp.float32),
        pltpu.VMEM((SELPAD,), jnp.float32),
        pltpu.VMEM((SELPAD,), jnp.float32),
        pltpu.VMEM((SELPAD,), jnp.float32),
        pltpu.VMEM((SELPAD,), jnp.float32),
        pltpu.VMEM((128,), jnp.float32),
        pltpu.VMEM((128,), jnp.int32),
        pltpu.VMEM((16,), jnp.int32),
    ],
)
def _nms_phase(boxes_t, scores_t, out_sc, out_ix,
               y1v, x1v, y2v, x2v, scv, gv, av,
               sy1, sx1, sy2, sx2, sar, osc, oix, ncell):
  wid = lax.axis_index("s") * 2 + lax.axis_index("c")
  lane = _lane()

  def reset_selected():
    zerov = jnp.zeros((16,), jnp.float32)
    for k in range(SELPAD // 16):
      sy1[pl.ds(k * 16, 16)] = zerov
      sx1[pl.ds(k * 16, 16)] = zerov
      sy2[pl.ds(k * 16, 16)] = zerov
      sx2[pl.ds(k * 16, 16)] = zerov
      sar[pl.ds(k * 16, 16)] = zerov

  def build_hierarchy():
    def buildg(g, _):
      base = lax.shift_left(g, 8)
      acc = scv[pl.ds(base, 16)]
      for kk in range(1, 16):
        acc = jnp.maximum(acc, scv[pl.ds(base + kk * 16, 16)])
      gv[pl.ds(g * 16, 16)] = acc
      return 0
    lax.fori_loop(0, NGRP, buildg, 0)

    def builda(k, _):
      acc = jnp.full((16,), NEG, jnp.float32)
      base = lax.shift_left(k, 8)
      for kk in range(16):
        acc = jnp.where(lane == kk, _bfly_max(gv[pl.ds(base + kk * 16, 16)]),
                        acc)
      av[pl.ds(k * 16, 16)] = acc
      return 0
    lax.fori_loop(0, NA // 16, builda, 0)

  def pop_top(aregs):
    t = jnp.maximum(jnp.maximum(aregs[0], aregs[1]),
                    jnp.maximum(aregs[2], aregs[3]))
    t = jnp.maximum(t, aregs[4])
    m = _bfly_max(t)
    ms = m[0]
    gacc = jnp.full((16,), BIGI, jnp.int32)
    for k in range(5):
      gacc = jnp.minimum(gacc, jnp.where(aregs[k] == m, k * 16 + lane, BIGI))
    gstar = _bfly_min(gacc)[0]
    base = lax.shift_left(gstar, 8)
    leafregs = [scv[pl.ds(base + j * 16, 16)] for j in range(16)]
    idxacc = jnp.full((16,), BIGI, jnp.int32)
    for j in range(16):
      idxacc = jnp.minimum(
          idxacc,
          jnp.where(leafregs[j] == m, base + j * 16 + lane, BIGI))
    idx = _bfly_min(idxacc)[0]
    return m, ms, gstar, idx, leafregs

  def mark_and_fix(aregs, gstar, idx, active, leafregs):
    cpos = lax.shift_right_logical(idx, 4) & 15
    tgt = jnp.where(active, idx & 15, jnp.int32(16))
    gmax = None
    newchunk = None
    for j in range(16):
      mj = jnp.where(cpos == j, jnp.float32(1.0), jnp.float32(0.0))
      nc = jnp.where(lane == tgt, jnp.float32(NEG), leafregs[j])
      blended = mj * nc + (1.0 - mj) * leafregs[j]
      gmax = blended if gmax is None else jnp.maximum(gmax, blended)
      newchunk = blended * mj if newchunk is None else (
          newchunk + blended * mj)
    scv[pl.ds(_align16(idx), 16)] = jnp.where(lane == tgt, jnp.float32(NEG),
                                              newchunk)
    gsel = jnp.where(active, gstar & 15, jnp.int32(16))
    gbase = _align16(gstar)
    gch = gv[pl.ds(gbase, 16)]
    gv[pl.ds(gbase, 16)] = jnp.where(lane == gsel, gmax, gch)
    anew = _bfly_max(gmax)
    out = []
    for k in range(5):
      tk = jnp.where(jnp.logical_and(active, lax.shift_right_logical(
          gstar, 4) == k), gstar & 15, jnp.int32(16))
      out.append(jnp.where(lane == tk, anew, aregs[k]))
    return tuple(out)

  def load_box(idx):
    base = _align16(idx)
    off = idx & 15
    by1 = _eread(y1v, base, off)
    bx1 = _eread(x1v, base, off)
    by2 = _eread(y2v, base, off)
    bx2 = _eread(x2v, base, off)
    return by1, bx1, by2, bx2, (by2 - by1) * (bx2 - bx1)

  def max_iou_vs_selected(by1, bx1, by2, bx2, barea):
    def iou_body(j, accf):
      ty1 = jnp.maximum(by1, sy1[pl.ds(j * 16, 16)])
      tx1 = jnp.maximum(bx1, sx1[pl.ds(j * 16, 16)])
      ty2 = jnp.minimum(by2, sy2[pl.ds(j * 16, 16)])
      tx2 = jnp.minimum(bx2, sx2[pl.ds(j * 16, 16)])
      inter = jnp.maximum(ty2 - ty1, 0.0) * jnp.maximum(tx2 - tx1, 0.0)
      iou = inter / (barea + sar[pl.ds(j * 16, 16)] - inter + 1e-8)
      return jnp.maximum(accf, iou)
    accf = lax.fori_loop(0, SELPAD // 16, iou_body,
                         jnp.zeros((16,), jnp.float32))
    return _bfly_max(accf)[0]

  def append(ns, rec, vsc, vix, bxs, add_sel):
    base = _align16(ns)
    tgt = jnp.where(rec, ns & 15, jnp.int32(16))
    _ewrite(osc, base, tgt, vsc)
    _ewrite(oix, base, tgt, vix)
    tgts = jnp.where(add_sel, ns & 15, jnp.int32(16))
    by1, bx1, by2, bx2, barea = bxs
    _ewrite(sy1, base, tgts, by1)
    _ewrite(sx1, base, tgts, bx1)
    _ewrite(sy2, base, tgts, by2)
    _ewrite(sx2, base, tgts, bx2)
    _ewrite(sar, base, tgts, barea)

  @pl.when(wid < NUM_CLASSES)
  def _():
    cls = wid
    pltpu.sync_copy(boxes_t.at[0], y1v)
    pltpu.sync_copy(boxes_t.at[1], x1v)
    pltpu.sync_copy(boxes_t.at[2], y2v)
    pltpu.sync_copy(boxes_t.at[3], x2v)
    pltpu.sync_copy(scores_t.at[cls], scv.at[pl.ds(0, NPAD)])

    negv = jnp.full((16,), NEG, jnp.float32)
    for k in range(NCHUNK, NSCV // 16):
      scv[pl.ds(k * 16, 16)] = negv
    padv = jnp.full((16,), PAD, jnp.float32)
    zeroiv = jnp.zeros((16,), jnp.int32)
    for k in range(8):
      osc[pl.ds(k * 16, 16)] = padv
      oix[pl.ds(k * 16, 16)] = zeroiv
    reset_selected()
    build_hierarchy()

    def walk_body(_, carry):
      ns = carry[0]
      aregs = carry[1:]
      m, ms, gstar, idx, leafregs = pop_top(aregs)
      valid = ms > jnp.float32(-0.5)
      done = ns < MAX_PER_CLASS
      active = jnp.logical_and(valid, done)
      bxs = load_box(idx)
      miou = max_iou_vs_selected(*bxs)
      accept = jnp.logical_and(active, miou <= IOU_THRESH)
      aregs = mark_and_fix(aregs, gstar, idx, active, leafregs)
      rec = jnp.logical_and(done, jnp.logical_or(accept,
                                                 jnp.logical_not(valid)))
      append(ns, rec, m, idx, bxs, accept)
      return (ns + jnp.where(rec, 1, 0),) + aregs

    ncell[pl.ds(0, 16)] = jnp.zeros((16,), jnp.int32)

    def walk_block(b, _):
      nsv = ncell[pl.ds(0, 16)]
      ns0 = nsv[0]

      @pl.when(ns0 < MAX_PER_CLASS)
      def _():
        aregs = tuple(av[pl.ds(k * 16, 16)] for k in range(5))
        carry = lax.fori_loop(0, 16, walk_body, (ns0,) + aregs)
        for k in range(5):
          av[pl.ds(k * 16, 16)] = carry[1 + k]
        ncell[pl.ds(0, 16)] = jnp.broadcast_to(carry[0], (16,))
      return 0

    lax.fori_loop(0, WALK_BLOCKS, walk_block, 0)
    ns_final = ncell[pl.ds(0, 16)][0]

    @pl.when(ns_final < MAX_PER_CLASS)
    def _():
      pltpu.sync_copy(scores_t.at[cls], scv.at[pl.ds(0, NPAD)])
      reset_selected()
      build_hierarchy()

      def eager_body(step, _):
        aregs = tuple(av[pl.ds(k * 16, 16)] for k in range(5))
        m, ms, gstar, idx, leafregs = pop_top(aregs)
        valid = ms > jnp.float32(-0.5)
        bxs = load_box(idx)
        by1, bx1, by2, bx2, barea = bxs
        append(step, True, m, idx, bxs, valid)

        def sweep(i, _):
          v = scv[pl.ds(i * 16, 16)]
          ty1 = jnp.maximum(by1, y1v[pl.ds(i * 16, 16)])
          tx1 = jnp.maximum(bx1, x1v[pl.ds(i * 16, 16)])
          ty2 = jnp.minimum(by2, y2v[pl.ds(i * 16, 16)])
          tx2 = jnp.minimum(bx2, x2v[pl.ds(i * 16, 16)])
          oy1 = y1v[pl.ds(i * 16, 16)]
          oarea = ((y2v[pl.ds(i * 16, 16)] - oy1) *
                   (x2v[pl.ds(i * 16, 16)] - x1v[pl.ds(i * 16, 16)]))
          inter = jnp.maximum(ty2 - ty1, 0.0) * jnp.maximum(tx2 - tx1, 0.0)
          iou = inter / (barea + oarea - inter + 1e-8)
          v = jnp.where(iou > IOU_THRESH, jnp.float32(NEG), v)
          v = jnp.where(i * 16 + _lane() == idx, jnp.float32(NEG), v)
          scv[pl.ds(i * 16, 16)] = v
          return 0
        lax.fori_loop(0, NCHUNK, sweep, 0)
        build_hierarchy()
        return 0

      lax.fori_loop(0, MAX_PER_CLASS, eager_body, 0)

    pltpu.sync_copy(osc, out_sc.at[cls])
    pltpu.sync_copy(oix, out_ix.at[cls])


@functools.partial(
    pl.kernel,
    out_type=[
        jax.ShapeDtypeStruct((BB_OUT,), jnp.float32),
        jax.ShapeDtypeStruct((CL_OUT,), jnp.int32),
    ],
    mesh=_MESH,
    scratch_types=[
        pltpu.VMEM((NB,), jnp.float32),
        pltpu.VMEM((NB,), jnp.int32),
        pltpu.VMEM((NPAD,), jnp.float32),
        pltpu.VMEM((NPAD,), jnp.float32),
        pltpu.VMEM((NPAD,), jnp.float32),
        pltpu.VMEM((NPAD,), jnp.float32),
        pltpu.VMEM((BB_OUT,), jnp.float32),
        pltpu.VMEM((CL_OUT,), jnp.int32),
    ],
)
def _topk_phase(flat_sc_h, flat_ix_h, boxes_t, out_bb, out_cl,
                fsc, fix, y1v, x1v, y2v, x2v, bbs, cls_s):
  wid = lax.axis_index("s") * 2 + lax.axis_index("c")
  lane = _lane()

  @pl.when(wid == 0)
  def _():
    pltpu.sync_copy(flat_sc_h, fsc)
    pltpu.sync_copy(flat_ix_h, fix)
    pltpu.sync_copy(boxes_t.at[0], y1v)
    pltpu.sync_copy(boxes_t.at[1], x1v)
    pltpu.sync_copy(boxes_t.at[2], y2v)
    pltpu.sync_copy(boxes_t.at[3], x2v)

    h0 = jnp.full((16,), LOW, jnp.float32)
    h1 = jnp.full((16,), LOW, jnp.float32)
    n0 = jnp.full((16,), LOW, jnp.float32)
    n1 = jnp.full((16,), LOW, jnp.float32)
    for c in range(16):
      h0 = jnp.where(lane == c, _eread(fsc, c * 128, 0), h0)
      n0 = jnp.where(lane == c, _eread(fsc, c * 128, 1), n0)
    for c in range(16, NUM_CLASSES):
      h1 = jnp.where(lane == (c - 16), _eread(fsc, c * 128, 0), h1)
      n1 = jnp.where(lane == (c - 16), _eread(fsc, c * 128, 1), n1)
    f0 = lane * 128
    f1 = (lane + 16) * 128

    def block(b, carry):
      h0, h1, n0, n1, f0, f1, clsacc = carry
      och = jnp.zeros((16,), jnp.float32)
      for s in range(4):
        r = 4 * b + s
        m = _bfly_max(jnp.maximum(h0, h1))
        facc = jnp.minimum(jnp.where(h0 == m, f0, BIGI),
                           jnp.where(h1 == m, f1, BIGI))
        fidx = _bfly_min(facc)[0]
        cls = lax.shift_right_logical(fidx, 7)

        h0 = jnp.where(f0 == fidx, n0, h0)
        h1 = jnp.where(f1 == fidx, n1, h1)
        nn = _eread(fsc, _align16(fidx + 2), (fidx + 2) & 15)
        n0 = jnp.where(f0 == fidx, nn, n0)
        n1 = jnp.where(f1 == fidx, nn, n1)
        f0 = jnp.where(f0 == fidx, f0 + 1, f0)
        f1 = jnp.where(f1 == fidx, f1 + 1, f1)

        ich = fix[pl.ds(_align16(fidx), 16)]
        bsel = jnp.where(lane == (fidx & 15), ich, jnp.int32(-1))
        bi = _bfly_max(bsel)[0]

        bbase = _align16(bi)
        boff = bi & 15
        och = jnp.where(lane == 4 * s + 0, _eread(y1v, bbase, boff), och)
        och = jnp.where(lane == 4 * s + 1, _eread(x1v, bbase, boff), och)
        och = jnp.where(lane == 4 * s + 2, _eread(y2v, bbase, boff), och)
        och = jnp.where(lane == 4 * s + 3, _eread(x2v, bbase, boff), och)
        clsacc = jnp.where(lane == (r & 15), cls, clsacc)

      bbs[pl.ds(b * 16, 16)] = och
      cls_s[pl.ds(_align16(b * 4), 16)] = clsacc
      return (h0, h1, n0, n1, f0, f1, clsacc)

    lax.fori_loop(0, MAX_PER_IMAGE // 4, block,
                  (h0, h1, n0, n1, f0, f1, jnp.zeros((16,), jnp.int32)))

    pltpu.sync_copy(bbs, out_bb)
    pltpu.sync_copy(cls_s, out_cl)


def kernel(boxes, scores):
  boxes_p = jnp.pad(boxes, ((0, NPAD - N), (0, 0)))
  scores_p = jnp.pad(scores, ((0, NPAD - N), (0, 0)), constant_values=NEG)
  boxes_t = boxes_p.T
  scores_t = scores_p.T
  sc_a, ix_a = _nms_phase(boxes_t, scores_t)
  bb_flat, cl = _topk_phase(sc_a.reshape(-1), ix_a.reshape(-1), boxes_t)
  return (bb_flat[:MAX_PER_IMAGE * 4].reshape(MAX_PER_IMAGE, 4),
          cl[:MAX_PER_IMAGE])


# speedup vs baseline: 49.2597x; 1.1115x over previous
import functools

import jax
import jax.numpy as jnp
from jax import lax
from jax.experimental import pallas as pl
from jax.experimental.pallas import tpu as pltpu
from jax.experimental.pallas import tpu_sc as plsc

N = 20000
NPAD = 20096
NSCV = 20480
NUM_CLASSES = 20
MAX_PER_CLASS = 100
MAX_PER_IMAGE = 300
IOU_THRESH = 0.7
NEG = -1e9
PAD = -2e9
LOW = -3e9
BIGI = jnp.int32(1 << 30)

NCHUNK = NPAD // 16
GRPCH = 8
NGRP = NSCV // (16 * GRPCH)
NA = NGRP
NAV = NA // 16
SELPAD = 112
WALK_BLOCKS = 14

NB = NUM_CLASSES * 128
BB_OUT = 1280
CL_OUT = 384

_MESH = plsc.VectorSubcoreMesh(
    core_axis_name="c", subcore_axis_name="s", num_cores=2, num_subcores=16
)


def _lane():
  return lax.iota(jnp.int32, 16)


def _bfly_max(v):
  lane = _lane()
  for sh in (8, 4, 2, 1):
    perm = lax.bitwise_xor(lane, jnp.int32(sh))
    v = jnp.maximum(v, v.at[perm].get(mode="promise_in_bounds"))
  return v


def _bfly_min(v):
  lane = _lane()
  for sh in (8, 4, 2, 1):
    perm = lax.bitwise_xor(lane, jnp.int32(sh))
    v = jnp.minimum(v, v.at[perm].get(mode="promise_in_bounds"))
  return v


def _eread(ref, base, off):
  ch = ref[pl.ds(base, 16)]
  return ch.at[jnp.broadcast_to(off, (16,))].get(mode="promise_in_bounds")


def _ewrite(ref, base, tgt_lane, val):
  ch = ref[pl.ds(base, 16)]
  ref[pl.ds(base, 16)] = jnp.where(_lane() == tgt_lane, val, ch)


def _align16(i):
  return lax.shift_left(lax.shift_right_logical(i, 4), 4)


@functools.partial(
    pl.kernel,
    out_type=[
        jax.ShapeDtypeStruct((NUM_CLASSES, 128), jnp.float32),
        jax.ShapeDtypeStruct((NUM_CLASSES, 128), jnp.int32),
    ],
    mesh=_MESH,
    scratch_types=[
        pltpu.VMEM((NPAD,), jnp.float32),
        pltpu.VMEM((NPAD,), jnp.float32),
        pltpu.VMEM((NPAD,), jnp.float32),
        pltpu.VMEM((NPAD,), jnp.float32),
        pltpu.VMEM((NSCV,), jnp.float32),
        pltpu.VMEM((NGRP * 16,), jnp.float32),
        pltpu.VMEM((NA,), jnp.float32),
        pltpu.VMEM((SELPAD,), jnp.float32),
        pltpu.VMEM((SELPAD,), jnp.float32),
        pltpu.VMEM((SELPAD,), jnp.float32),
        pltpu.VMEM((SELPAD,), jnp.float32),
        pltpu.VMEM((SELPAD,), jnp.float32),
        pltpu.VMEM((128,), jnp.float32),
        pltpu.VMEM((128,), jnp.int32),
        pltpu.VMEM((16,), jnp.int32),
        pltpu.SemaphoreType.DMA,
    ],
)
def _nms_phase(boxes_t, scores_t, out_sc, out_ix,
               y1v, x1v, y2v, x2v, scv, gv, av,
               sy1, sx1, sy2, sx2, sar, osc, oix, ncell, dsem):
  wid = lax.axis_index("s") * 2 + lax.axis_index("c")
  lane = _lane()

  def reset_selected():
    zerov = jnp.zeros((16,), jnp.float32)
    for k in range(SELPAD // 16):
      sy1[pl.ds(k * 16, 16)] = zerov
      sx1[pl.ds(k * 16, 16)] = zerov
      sy2[pl.ds(k * 16, 16)] = zerov
      sx2[pl.ds(k * 16, 16)] = zerov
      sar[pl.ds(k * 16, 16)] = zerov

  def build_hierarchy():
    def buildg(g, _):
      base = lax.shift_left(g, 7)
      acc = scv[pl.ds(base, 16)]
      for kk in range(1, GRPCH):
        acc = jnp.maximum(acc, scv[pl.ds(base + kk * 16, 16)])
      gv[pl.ds(g * 16, 16)] = acc
      return 0
    lax.fori_loop(0, NGRP, buildg, 0)

    def builda(k, _):
      acc = jnp.full((16,), NEG, jnp.float32)
      base = lax.shift_left(k, 8)
      for kk in range(16):
        acc = jnp.where(lane == kk, _bfly_max(gv[pl.ds(base + kk * 16, 16)]),
                        acc)
      av[pl.ds(k * 16, 16)] = acc
      return 0
    lax.fori_loop(0, NAV, builda, 0)

  def pop_top(aregs):
    t = aregs[0]
    for k in range(1, NAV):
      t = jnp.maximum(t, aregs[k])
    m = _bfly_max(t)
    ms = m[0]
    gacc = jnp.full((16,), BIGI, jnp.int32)
    for k in range(NAV):
      gacc = jnp.minimum(gacc, jnp.where(aregs[k] == m, k * 16 + lane, BIGI))
    gstar = _bfly_min(gacc)[0]
    base = lax.shift_left(gstar, 7)
    leafregs = [scv[pl.ds(base + j * 16, 16)] for j in range(GRPCH)]
    idxacc = jnp.full((16,), BIGI, jnp.int32)
    for j in range(GRPCH):
      idxacc = jnp.minimum(
          idxacc,
          jnp.where(leafregs[j] == m, base + j * 16 + lane, BIGI))
    idx = _bfly_min(idxacc)[0]
    return m, ms, gstar, idx, leafregs

  def mark_and_fix(aregs, gstar, idx, active, leafregs):
    cpos = lax.shift_right_logical(idx, 4) & (GRPCH - 1)
    tgt = jnp.where(active, idx & 15, jnp.int32(16))
    gmax = None
    newchunk = None
    for j in range(GRPCH):
      mj = jnp.where(cpos == j, jnp.float32(1.0), jnp.float32(0.0))
      nc = jnp.where(lane == tgt, jnp.float32(NEG), leafregs[j])
      blended = mj * nc + (1.0 - mj) * leafregs[j]
      gmax = blended if gmax is None else jnp.maximum(gmax, blended)
      newchunk = blended * mj if newchunk is None else (
          newchunk + blended * mj)
    scv[pl.ds(_align16(idx), 16)] = jnp.where(lane == tgt, jnp.float32(NEG),
                                              newchunk)
    gsel = jnp.where(active, gstar & 15, jnp.int32(16))
    gbase = _align16(gstar)
    gch = gv[pl.ds(gbase, 16)]
    gv[pl.ds(gbase, 16)] = jnp.where(lane == gsel, gmax, gch)
    anew = _bfly_max(gmax)
    out = []
    for k in range(NAV):
      tk = jnp.where(jnp.logical_and(active, lax.shift_right_logical(
          gstar, 4) == k), gstar & 15, jnp.int32(16))
      out.append(jnp.where(lane == tk, anew, aregs[k]))
    return tuple(out)

  def load_box(idx):
    base = _align16(idx)
    off = idx & 15
    by1 = _eread(y1v, base, off)
    bx1 = _eread(x1v, base, off)
    by2 = _eread(y2v, base, off)
    bx2 = _eread(x2v, base, off)
    return by1, bx1, by2, bx2, (by2 - by1) * (bx2 - bx1)

  def suppressed_by_selected(by1, bx1, by2, bx2, barea):
    accf = jnp.full((16,), -1.0, jnp.float32)
    for j in range(SELPAD // 16):
      vy1 = sy1[pl.ds(j * 16, 16)]
      vx1 = sx1[pl.ds(j * 16, 16)]
      vy2 = sy2[pl.ds(j * 16, 16)]
      vx2 = sx2[pl.ds(j * 16, 16)]
      ty1 = jnp.maximum(by1, vy1)
      tx1 = jnp.maximum(bx1, vx1)
      ty2 = jnp.minimum(by2, vy2)
      tx2 = jnp.minimum(bx2, vx2)
      inter = jnp.maximum(ty2 - ty1, 0.0) * jnp.maximum(tx2 - tx1, 0.0)
      denom = barea + (vy2 - vy1) * (vx2 - vx1) - inter + 1e-8
      accf = jnp.maximum(accf, inter - IOU_THRESH * denom)
    return _bfly_max(accf)[0] > jnp.float32(0.0)

  def append(ns, rec, vsc, vix, bxs, add_sel):
    base = _align16(ns)
    tgt = jnp.where(rec, ns & 15, jnp.int32(16))
    _ewrite(osc, base, tgt, vsc)
    _ewrite(oix, base, tgt, vix)
    tgts = jnp.where(add_sel, ns & 15, jnp.int32(16))
    by1, bx1, by2, bx2, barea = bxs
    _ewrite(sy1, base, tgts, by1)
    _ewrite(sx1, base, tgts, bx1)
    _ewrite(sy2, base, tgts, by2)
    _ewrite(sx2, base, tgts, bx2)
    _ewrite(sar, base, tgts, barea)

  @pl.when(wid < NUM_CLASSES)
  def _():
    cls = wid
    cps = [pltpu.async_copy(boxes_t.at[0], y1v, dsem),
           pltpu.async_copy(boxes_t.at[1], x1v, dsem),
           pltpu.async_copy(boxes_t.at[2], y2v, dsem),
           pltpu.async_copy(boxes_t.at[3], x2v, dsem),
           pltpu.async_copy(scores_t.at[cls], scv.at[pl.ds(0, NPAD)], dsem)]
    for cp in cps:
      cp.wait()

    negv = jnp.full((16,), NEG, jnp.float32)
    for k in range(NCHUNK, NSCV // 16):
      scv[pl.ds(k * 16, 16)] = negv
    padv = jnp.full((16,), PAD, jnp.float32)
    zeroiv = jnp.zeros((16,), jnp.int32)
    for k in range(8):
      osc[pl.ds(k * 16, 16)] = padv
      oix[pl.ds(k * 16, 16)] = zeroiv
    reset_selected()
    build_hierarchy()

    def walk_body(_, carry):
      ns = carry[0]
      aregs = carry[1:]
      m, ms, gstar, idx, leafregs = pop_top(aregs)
      valid = ms > jnp.float32(-0.5)
      done = ns < MAX_PER_CLASS
      bxs = load_box(idx)
      sup = suppressed_by_selected(*bxs)
      accept = jnp.logical_and(jnp.logical_and(valid, done),
                               jnp.logical_not(sup))
      aregs = mark_and_fix(aregs, gstar, idx, valid, leafregs)
      rec = jnp.logical_and(done, jnp.logical_or(accept,
                                                 jnp.logical_not(valid)))
      append(ns, rec, m, idx, bxs, accept)
      return (ns + jnp.where(rec, 1, 0),) + aregs

    ncell[pl.ds(0, 16)] = jnp.zeros((16,), jnp.int32)

    def walk_block(b, _):
      nsv = ncell[pl.ds(0, 16)]
      ns0 = nsv[0]

      @pl.when(ns0 < MAX_PER_CLASS)
      def _():
        aregs = tuple(av[pl.ds(k * 16, 16)] for k in range(NAV))
        carry = lax.fori_loop(0, 16, walk_body, (ns0,) + aregs)
        for k in range(NAV):
          av[pl.ds(k * 16, 16)] = carry[1 + k]
        ncell[pl.ds(0, 16)] = jnp.broadcast_to(carry[0], (16,))
      return 0

    lax.fori_loop(0, WALK_BLOCKS, walk_block, 0)
    ns_final = ncell[pl.ds(0, 16)][0]

    @pl.when(ns_final < MAX_PER_CLASS)
    def _():
      pltpu.sync_copy(scores_t.at[cls], scv.at[pl.ds(0, NPAD)])
      reset_selected()
      build_hierarchy()

      def eager_body(step, _):
        aregs = tuple(av[pl.ds(k * 16, 16)] for k in range(NAV))
        m, ms, gstar, idx, leafregs = pop_top(aregs)
        valid = ms > jnp.float32(-0.5)
        bxs = load_box(idx)
        by1, bx1, by2, bx2, barea = bxs
        append(step, True, m, idx, bxs, valid)

        def sweep(i, _):
          v = scv[pl.ds(i * 16, 16)]
          ty1 = jnp.maximum(by1, y1v[pl.ds(i * 16, 16)])
          tx1 = jnp.maximum(bx1, x1v[pl.ds(i * 16, 16)])
          ty2 = jnp.minimum(by2, y2v[pl.ds(i * 16, 16)])
          tx2 = jnp.minimum(bx2, x2v[pl.ds(i * 16, 16)])
          oy1 = y1v[pl.ds(i * 16, 16)]
          oarea = ((y2v[pl.ds(i * 16, 16)] - oy1) *
                   (x2v[pl.ds(i * 16, 16)] - x1v[pl.ds(i * 16, 16)]))
          inter = jnp.maximum(ty2 - ty1, 0.0) * jnp.maximum(tx2 - tx1, 0.0)
          iou = inter / (barea + oarea - inter + 1e-8)
          v = jnp.where(iou > IOU_THRESH, jnp.float32(NEG), v)
          v = jnp.where(i * 16 + _lane() == idx, jnp.float32(NEG), v)
          scv[pl.ds(i * 16, 16)] = v
          return 0
        lax.fori_loop(0, NCHUNK, sweep, 0)
        build_hierarchy()
        return 0

      lax.fori_loop(0, MAX_PER_CLASS, eager_body, 0)

    pltpu.sync_copy(osc, out_sc.at[cls])
    pltpu.sync_copy(oix, out_ix.at[cls])


@functools.partial(
    pl.kernel,
    out_type=[
        jax.ShapeDtypeStruct((BB_OUT,), jnp.float32),
        jax.ShapeDtypeStruct((CL_OUT,), jnp.int32),
    ],
    mesh=_MESH,
    scratch_types=[
        pltpu.VMEM((NB,), jnp.float32),
        pltpu.VMEM((NB,), jnp.int32),
        pltpu.VMEM((NPAD,), jnp.float32),
        pltpu.VMEM((NPAD,), jnp.float32),
        pltpu.VMEM((NPAD,), jnp.float32),
        pltpu.VMEM((NPAD,), jnp.float32),
        pltpu.VMEM((BB_OUT,), jnp.float32),
        pltpu.VMEM((CL_OUT,), jnp.int32),
        pltpu.SemaphoreType.DMA,
    ],
)
def _topk_phase(flat_sc_h, flat_ix_h, boxes_t, out_bb, out_cl,
                fsc, fix, y1v, x1v, y2v, x2v, bbs, cls_s, dsem):
  wid = lax.axis_index("s") * 2 + lax.axis_index("c")
  lane = _lane()

  @pl.when(wid == 0)
  def _():
    cps = [pltpu.async_copy(flat_sc_h, fsc, dsem),
           pltpu.async_copy(flat_ix_h, fix, dsem),
           pltpu.async_copy(boxes_t.at[0], y1v, dsem),
           pltpu.async_copy(boxes_t.at[1], x1v, dsem),
           pltpu.async_copy(boxes_t.at[2], y2v, dsem),
           pltpu.async_copy(boxes_t.at[3], x2v, dsem)]
    for cp in cps:
      cp.wait()

    h0 = jnp.full((16,), LOW, jnp.float32)
    h1 = jnp.full((16,), LOW, jnp.float32)
    n0 = jnp.full((16,), LOW, jnp.float32)
    n1 = jnp.full((16,), LOW, jnp.float32)
    for c in range(16):
      h0 = jnp.where(lane == c, _eread(fsc, c * 128, 0), h0)
      n0 = jnp.where(lane == c, _eread(fsc, c * 128, 1), n0)
    for c in range(16, NUM_CLASSES):
      h1 = jnp.where(lane == (c - 16), _eread(fsc, c * 128, 0), h1)
      n1 = jnp.where(lane == (c - 16), _eread(fsc, c * 128, 1), n1)
    f0 = lane * 128
    f1 = (lane + 16) * 128

    def block(b, carry):
      h0, h1, n0, n1, f0, f1, clsacc = carry
      och = jnp.zeros((16,), jnp.float32)
      for s in range(4):
        r = 4 * b + s
        m = _bfly_max(jnp.maximum(h0, h1))
        facc = jnp.minimum(jnp.where(h0 == m, f0, BIGI),
                           jnp.where(h1 == m, f1, BIGI))
        fidx = _bfly_min(facc)[0]
        cls = lax.shift_right_logical(fidx, 7)

        h0 = jnp.where(f0 == fidx, n0, h0)
        h1 = jnp.where(f1 == fidx, n1, h1)
        nn = _eread(fsc, _align16(fidx + 2), (fidx + 2) & 15)
        n0 = jnp.where(f0 == fidx, nn, n0)
        n1 = jnp.where(f1 == fidx, nn, n1)
        f0 = jnp.where(f0 == fidx, f0 + 1, f0)
        f1 = jnp.where(f1 == fidx, f1 + 1, f1)

        ich = fix[pl.ds(_align16(fidx), 16)]
        bsel = jnp.where(lane == (fidx & 15), ich, jnp.int32(-1))
        bi = _bfly_max(bsel)[0]

        bbase = _align16(bi)
        boff = bi & 15
        och = jnp.where(lane == 4 * s + 0, _eread(y1v, bbase, boff), och)
        och = jnp.where(lane == 4 * s + 1, _eread(x1v, bbase, boff), och)
        och = jnp.where(lane == 4 * s + 2, _eread(y2v, bbase, boff), och)
        och = jnp.where(lane == 4 * s + 3, _eread(x2v, bbase, boff), och)
        clsacc = jnp.where(lane == (r & 15), cls, clsacc)

      bbs[pl.ds(b * 16, 16)] = och
      cls_s[pl.ds(_align16(b * 4), 16)] = clsacc
      return (h0, h1, n0, n1, f0, f1, clsacc)

    lax.fori_loop(0, MAX_PER_IMAGE // 4, block,
                  (h0, h1, n0, n1, f0, f1, jnp.zeros((16,), jnp.int32)))

    pltpu.sync_copy(bbs, out_bb)
    pltpu.sync_copy(cls_s, out_cl)


def kernel(boxes, scores):
  boxes_p = jnp.pad(boxes, ((0, NPAD - N), (0, 0)))
  scores_p = jnp.pad(scores, ((0, NPAD - N), (0, 0)), constant_values=NEG)
  boxes_t = boxes_p.T
  scores_t = scores_p.T
  sc_a, ix_a = _nms_phase(boxes_t, scores_t)
  bb_flat, cl = _topk_phase(sc_a.reshape(-1), ix_a.reshape(-1), boxes_t)
  return (bb_flat[:MAX_PER_IMAGE * 4].reshape(MAX_PER_IMAGE, 4),
          cl[:MAX_PER_IMAGE])
